# baseline (speedup 1.0000x reference)
import jax
import jax.numpy as jnp
from jax import lax
from jax.experimental import pallas as pl
from jax.experimental.pallas import tpu as pltpu
from jax.experimental.pallas import tpu_sc as plsc

VOCAB = 100000
EMBED_DIM = 64
MAX_SEQ_LEN = 512
BATCH = 1024
SEQ_LEN = 200
LN_EPS = 1e-5

NUM_CORES = 2
NUM_SUBCORES = 16
LANES = 16
NUM_WORKERS = NUM_CORES * NUM_SUBCORES
ROWS_PER_WORKER = BATCH // NUM_WORKERS
VPR = EMBED_DIM // LANES
SPLIT = 104

_GATHER_DNUMS = lax.GatherDimensionNumbers(
    offset_dims=(), collapsed_slice_dims=(0,), start_index_map=(0,))


def _lane_permute(v, p):
    return lax.gather(v, p[:, None], _GATHER_DNUMS, slice_sizes=(1,),
                      mode=lax.GatherScatterMode.PROMISE_IN_BOUNDS)


def _splat_sum(v, perms):
    for p in perms:
        v = v + _lane_permute(v, p)
    return v


def _ln_body(ids_hbm, token_table, pos_hbm, gamma_hbm, beta_hbm, out_hbm,
             idx_bufs, rows_bufs, out_bufs, pos_slab, gamma_v, beta_v,
             gsems, osems, isems):
    wid = lax.axis_index("s") * NUM_CORES + lax.axis_index("c")
    base = wid * ROWS_PER_WORKER

    pltpu.sync_copy(pos_hbm.at[pl.ds(0, SEQ_LEN)], pos_slab)
    pltpu.sync_copy(gamma_hbm, gamma_v)
    pltpu.sync_copy(beta_hbm, beta_v)

    iota = lax.iota(jnp.int32, LANES)
    perms = [iota ^ (1 << k) for k in range(4)]

    gammas = [gamma_v[pl.ds(k * LANES, LANES)] for k in range(VPR)]
    betas = [beta_v[pl.ds(k * LANES, LANES)] for k in range(VPR)]
    inv_d = jnp.float32(1.0 / EMBED_DIM)

    GRP = 4

    def normalize_row(rows_buf, out_buf):
        def grp_body(g, carry):
            r0 = g * GRP
            es = [[rows_buf[r0 + j, pl.ds(k * LANES, LANES)] +
                   pos_slab[r0 + j, pl.ds(k * LANES, LANES)]
                   for k in range(VPR)] for j in range(GRP)]
            ts = [(e[0] + e[1]) + (e[2] + e[3]) for e in es]
            us = [(e[0] * e[0] + e[1] * e[1]) + (e[2] * e[2] + e[3] * e[3])
                  for e in es]
            for p in perms:
                ts = [t + _lane_permute(t, p) for t in ts]
                us = [u + _lane_permute(u, p) for u in us]
            means = [t * inv_d for t in ts]
            vrs = [u * inv_d - m * m + jnp.float32(LN_EPS)
                   for u, m in zip(us, means)]
            ys = [lax.bitcast_convert_type(
                      jnp.int32(0x5F3759DF)
                      - (lax.bitcast_convert_type(v, jnp.int32) >> 1),
                      jnp.float32) for v in vrs]
            hs = [v * jnp.float32(0.5) for v in vrs]
            yys = [y * y for y in ys]
            ys = [y * (jnp.float32(1.5) - h * yy)
                  for y, h, yy in zip(ys, hs, yys)]
            sms = [m * y for m, y in zip(means, ys)]
            for k in range(VPR):
                for j in range(GRP):
                    out_buf[g * (GRP // 2) + j // 2,
                            pl.ds((j % 2) * EMBED_DIM + k * LANES, LANES)] = (
                        (es[j][k] * ys[j] - sms[j]) * gammas[k] + betas[k])
            return carry

        lax.fori_loop(0, SEQ_LEN // GRP, grp_body, 0)

    def start_gather(slot):
        pltpu.async_copy(token_table.at[idx_bufs[slot].at[pl.ds(0, SPLIT)]],
                         rows_bufs[slot].at[pl.ds(0, SPLIT)], gsems[slot])
        pltpu.async_copy(
            token_table.at[idx_bufs[slot].at[pl.ds(SPLIT, SEQ_LEN - SPLIT)]],
            rows_bufs[slot].at[pl.ds(SPLIT, SEQ_LEN - SPLIT)], gsems[slot])

    def wait_gather(slot):
        pltpu.make_async_copy(
            token_table.at[idx_bufs[slot].at[pl.ds(0, SPLIT)]],
            rows_bufs[slot].at[pl.ds(0, SPLIT)], gsems[slot]).wait()
        pltpu.make_async_copy(
            token_table.at[idx_bufs[slot].at[pl.ds(SPLIT, SEQ_LEN - SPLIT)]],
            rows_bufs[slot].at[pl.ds(SPLIT, SEQ_LEN - SPLIT)],
            gsems[slot]).wait()


    for slot in range(2):
        pltpu.sync_copy(ids_hbm.at[base + slot], idx_bufs[slot])
        start_gather(slot)

    def row_step(cc, carry):
        for k in range(2):
            b = base + cc * 2 + k
            prefetch = cc < (ROWS_PER_WORKER // 2) - 1

            wait_gather(k)

            @pl.when(prefetch)
            def _():
                pltpu.async_copy(ids_hbm.at[b + 2], idx_bufs[k], isems[k])

            @pl.when(cc > 0)
            def _():
                pltpu.make_async_copy(
                    out_bufs[k],
                    out_hbm.at[pl.ds((b - 2) * (SEQ_LEN // 2), SEQ_LEN // 2)],
                    osems[k]).wait()

            normalize_row(rows_bufs[k], out_bufs[k])

            pltpu.async_copy(
                out_bufs[k],
                out_hbm.at[pl.ds(b * (SEQ_LEN // 2), SEQ_LEN // 2)], osems[k])

            @pl.when(prefetch)
            def _():
                pltpu.make_async_copy(ids_hbm.at[b + 2], idx_bufs[k],
                                      isems[k]).wait()
                start_gather(k)
        return carry

    lax.fori_loop(0, ROWS_PER_WORKER // 2, row_step, 0)

    for k in range(2):
        b = base + ROWS_PER_WORKER - 2 + k
        pltpu.make_async_copy(
            out_bufs[k], out_hbm.at[pl.ds(b * (SEQ_LEN // 2), SEQ_LEN // 2)],
            osems[k]).wait()


@jax.jit
def _hybrid_embed(token_ids, token_table, pos_table, ln_gamma, ln_beta):
    mesh = plsc.VectorSubcoreMesh(core_axis_name="c", subcore_axis_name="s",
                                  num_cores=NUM_CORES,
                                  num_subcores=NUM_SUBCORES)
    return pl.kernel(
        _ln_body,
        out_type=jax.ShapeDtypeStruct((BATCH * SEQ_LEN // 2, 2 * EMBED_DIM),
                                      jnp.float32),
        mesh=mesh,
        scratch_types=[
            [pltpu.VMEM((SEQ_LEN,), jnp.int32) for _ in range(2)],
            [pltpu.VMEM((SEQ_LEN, EMBED_DIM), jnp.float32) for _ in range(2)],
            [pltpu.VMEM((SEQ_LEN // 2, 2 * EMBED_DIM), jnp.float32)
             for _ in range(2)],
            pltpu.VMEM((SEQ_LEN, EMBED_DIM), jnp.float32),
            pltpu.VMEM((EMBED_DIM,), jnp.float32),
            pltpu.VMEM((EMBED_DIM,), jnp.float32),
            [pltpu.SemaphoreType.DMA for _ in range(2)],
            [pltpu.SemaphoreType.DMA for _ in range(2)],
            [pltpu.SemaphoreType.DMA for _ in range(2)],
        ],
        compiler_params=pltpu.CompilerParams(use_tc_tiling_on_sc=False),
    )(token_ids, token_table, pos_table, ln_gamma, ln_beta)


def kernel(token_ids, token_table, pos_table, ln_gamma, ln_beta):
    out = _hybrid_embed(token_ids.astype(jnp.int32), token_table, pos_table,
                        ln_gamma, ln_beta)
    return out.reshape(BATCH, SEQ_LEN, EMBED_DIM)


# speedup vs baseline: 1.3771x; 1.3771x over previous
import jax
import jax.numpy as jnp
from jax import lax
from jax.experimental import pallas as pl
from jax.experimental.pallas import tpu as pltpu
from jax.experimental.pallas import tpu_sc as plsc

VOCAB = 100000
EMBED_DIM = 64
MAX_SEQ_LEN = 512
BATCH = 1024
SEQ_LEN = 200
LN_EPS = 1e-5

NUM_CORES = 2
NUM_SUBCORES = 16
LANES = 16
NUM_WORKERS = NUM_CORES * NUM_SUBCORES
ROWS_PER_WORKER = BATCH // NUM_WORKERS
VPR = EMBED_DIM // LANES
SPLIT = 104

_GATHER_DNUMS = lax.GatherDimensionNumbers(
    offset_dims=(), collapsed_slice_dims=(0,), start_index_map=(0,))


def _lane_permute(v, p):
    return lax.gather(v, p[:, None], _GATHER_DNUMS, slice_sizes=(1,),
                      mode=lax.GatherScatterMode.PROMISE_IN_BOUNDS)


def _splat_sum(v, perms):
    for p in perms:
        v = v + _lane_permute(v, p)
    return v


def _ln_body(ids_hbm, token_table, pos_hbm, gamma_hbm, beta_hbm, out_hbm,
             idx_bufs, rows_bufs, out_bufs, pos_slab, gamma_v, beta_v,
             gsems, osems, isems):
    wid = lax.axis_index("s") * NUM_CORES + lax.axis_index("c")
    base = wid * ROWS_PER_WORKER

    pltpu.sync_copy(pos_hbm.at[pl.ds(0, SEQ_LEN)], pos_slab)
    pltpu.sync_copy(gamma_hbm, gamma_v)
    pltpu.sync_copy(beta_hbm, beta_v)

    iota = lax.iota(jnp.int32, LANES)
    perms = [iota ^ (1 << k) for k in range(4)]

    gammas = [gamma_v[pl.ds(k * LANES, LANES)] for k in range(VPR)]
    betas = [beta_v[pl.ds(k * LANES, LANES)] for k in range(VPR)]
    inv_d = jnp.float32(1.0 / EMBED_DIM)

    GRP = 4

    def normalize_row(rows_buf, out_buf):
        def grp_body(g, carry):
            r0 = g * GRP
            es = [[rows_buf[r0 + j, pl.ds(k * LANES, LANES)] +
                   pos_slab[r0 + j, pl.ds(k * LANES, LANES)]
                   for k in range(VPR)] for j in range(GRP)]
            ts = [(e[0] + e[1]) + (e[2] + e[3]) for e in es]
            us = [(e[0] * e[0] + e[1] * e[1]) + (e[2] * e[2] + e[3] * e[3])
                  for e in es]
            for p in perms:
                ts = [t + _lane_permute(t, p) for t in ts]
                us = [u + _lane_permute(u, p) for u in us]
            means = [t * inv_d for t in ts]
            vrs = [u * inv_d - m * m + jnp.float32(LN_EPS)
                   for u, m in zip(us, means)]
            ys = [lax.bitcast_convert_type(
                      jnp.int32(0x5F3759DF)
                      - (lax.bitcast_convert_type(v, jnp.int32) >> 1),
                      jnp.float32) for v in vrs]
            hs = [v * jnp.float32(0.5) for v in vrs]
            yys = [y * y for y in ys]
            ys = [y * (jnp.float32(1.5) - h * yy)
                  for y, h, yy in zip(ys, hs, yys)]
            sms = [m * y for m, y in zip(means, ys)]
            for k in range(VPR):
                for j in range(GRP):
                    out_buf[r0 + j, pl.ds(k * LANES, LANES)] = (
                        (es[j][k] * ys[j] - sms[j]) * gammas[k] + betas[k])
            return carry

        lax.fori_loop(0, SEQ_LEN // GRP, grp_body, 0)

    def start_gather(slot):
        pltpu.async_copy(token_table.at[idx_bufs[slot].at[pl.ds(0, SPLIT)]],
                         rows_bufs[slot].at[pl.ds(0, SPLIT)], gsems[slot])
        pltpu.async_copy(
            token_table.at[idx_bufs[slot].at[pl.ds(SPLIT, SEQ_LEN - SPLIT)]],
            rows_bufs[slot].at[pl.ds(SPLIT, SEQ_LEN - SPLIT)], gsems[slot])

    def wait_gather(slot):
        pltpu.make_async_copy(
            token_table.at[idx_bufs[slot].at[pl.ds(0, SPLIT)]],
            rows_bufs[slot].at[pl.ds(0, SPLIT)], gsems[slot]).wait()
        pltpu.make_async_copy(
            token_table.at[idx_bufs[slot].at[pl.ds(SPLIT, SEQ_LEN - SPLIT)]],
            rows_bufs[slot].at[pl.ds(SPLIT, SEQ_LEN - SPLIT)],
            gsems[slot]).wait()


    for slot in range(2):
        pltpu.sync_copy(ids_hbm.at[base + slot], idx_bufs[slot])
        start_gather(slot)

    def row_step(cc, carry):
        for k in range(2):
            b = base + cc * 2 + k
            prefetch = cc < (ROWS_PER_WORKER // 2) - 1

            wait_gather(k)

            @pl.when(prefetch)
            def _():
                pltpu.async_copy(ids_hbm.at[b + 2], idx_bufs[k], isems[k])

            @pl.when(cc > 0)
            def _():
                pltpu.make_async_copy(
                    out_bufs[k],
                    out_hbm.at[b - 2, :, pl.ds(0, EMBED_DIM)],
                    osems[k]).wait()

            normalize_row(rows_bufs[k], out_bufs[k])

            pltpu.async_copy(out_bufs[k],
                             out_hbm.at[b, :, pl.ds(0, EMBED_DIM)], osems[k])

            @pl.when(prefetch)
            def _():
                pltpu.make_async_copy(ids_hbm.at[b + 2], idx_bufs[k],
                                      isems[k]).wait()
                start_gather(k)
        return carry

    lax.fori_loop(0, ROWS_PER_WORKER // 2, row_step, 0)

    for k in range(2):
        b = base + ROWS_PER_WORKER - 2 + k
        pltpu.make_async_copy(out_bufs[k],
                              out_hbm.at[b, :, pl.ds(0, EMBED_DIM)],
                              osems[k]).wait()


@jax.jit
def _hybrid_embed(token_ids, token_table, pos_table, ln_gamma, ln_beta):
    mesh = plsc.VectorSubcoreMesh(core_axis_name="c", subcore_axis_name="s",
                                  num_cores=NUM_CORES,
                                  num_subcores=NUM_SUBCORES)
    return pl.kernel(
        _ln_body,
        out_type=jax.ShapeDtypeStruct((BATCH, SEQ_LEN, 2 * EMBED_DIM),
                                      jnp.float32),
        mesh=mesh,
        scratch_types=[
            [pltpu.VMEM((SEQ_LEN,), jnp.int32) for _ in range(2)],
            [pltpu.VMEM((SEQ_LEN, EMBED_DIM), jnp.float32) for _ in range(2)],
            [pltpu.VMEM((SEQ_LEN, EMBED_DIM), jnp.float32) for _ in range(2)],
            pltpu.VMEM((SEQ_LEN, EMBED_DIM), jnp.float32),
            pltpu.VMEM((EMBED_DIM,), jnp.float32),
            pltpu.VMEM((EMBED_DIM,), jnp.float32),
            [pltpu.SemaphoreType.DMA for _ in range(2)],
            [pltpu.SemaphoreType.DMA for _ in range(2)],
            [pltpu.SemaphoreType.DMA for _ in range(2)],
        ],
        compiler_params=pltpu.CompilerParams(use_tc_tiling_on_sc=False),
    )(token_ids, token_table, pos_table, ln_gamma, ln_beta)


def kernel(token_ids, token_table, pos_table, ln_gamma, ln_beta):
    out = _hybrid_embed(token_ids.astype(jnp.int32), token_table, pos_table,
                        ln_gamma, ln_beta)
    return out[..., :EMBED_DIM]
